# baseline (speedup 1.0000x reference)
import functools

import jax
import jax.numpy as jnp
from jax import lax
from jax.experimental import pallas as pl
from jax.experimental.pallas import tpu as pltpu
from jax.experimental.pallas import tpu_sc as plsc

N_NODES = 10000
N_EDGES = 320000
NUM_REL = 256
D = 128
H = 8
DH = 16
HOPS = 3
ALPHA = 0.15

NC = 2
NS = 16
L = 16

CH = 80
EPT = N_EDGES // NS
NCHUNK = EPT // CH
NPAD = 10240
NPT = NPAD // NS
RU = 64
NU = NPT // RU
CPH = 4
NPAIR = NCHUNK // 2


def _lane_iota():
    return lax.iota(jnp.int32, L)


def _splat(x):
    return jnp.full((L,), x, jnp.int32)


def _sc_layer_body(with_rel, src_h, dst_h, et_h, q_h, k_h, v_h, relp_h, z64_h,
                   z16_h, feat_o, ex_o,
                   feat_s, agg_s, den_s, relp_s,
                   g0, g1, ex0, ex1, src0, src1, dst0, dst1,
                   q_t, rel_t, et_t, et1_t, ua_t, ud_t, uv_t,
                   semi0, semi1, semg0, semg1, sem, semi2, semi3, semi4, semi5):
    c = lax.axis_index("c")
    s = lax.axis_index("s")
    ebase = s * EPT
    nbase = s * NPT
    lane = _lane_iota()

    qc = q_h.at[c]
    kc = k_h.at[c]
    vc = v_h.at[c]
    fo = feat_o.at[c]
    exc = ex_o.at[c]

    for u in range(NU):
        rb = nbase + u * RU
        pltpu.sync_copy(vc.at[pl.ds(rb, RU)], uv_t)
        pltpu.sync_copy(uv_t, feat_s.at[pl.ds(rb, RU)])
        pltpu.sync_copy(z16_h.at[pl.ds(u * RU, RU)], ud_t)
        pltpu.sync_copy(ud_t, den_s.at[pl.ds(rb, RU)])
    if with_rel:
        @pl.when(s == 0)
        def _copy_relp():
            for j in range(NUM_REL // RU):
                pltpu.sync_copy(relp_h.at[c, pl.ds(j * RU, RU)], ua_t)
                pltpu.sync_copy(ua_t, relp_s.at[pl.ds(j * RU, RU)])
    plsc.subcore_barrier()

    def issue_bidx(ci, sv, dv, ev, sems):
        off = ebase + ci * CH
        pltpu.async_copy(src_h.at[pl.ds(off, CH)], sv, sems[0])
        pltpu.async_copy(dst_h.at[pl.ds(off, CH)], dv, sems[1])
        if with_rel:
            pltpu.async_copy(et_h.at[pl.ds(off, CH)], ev, sems[2])

    def wait_bidx(ci, sv, dv, ev, sems):
        off = ebase + ci * CH
        pltpu.make_async_copy(src_h.at[pl.ds(off, CH)], sv, sems[0]).wait()
        pltpu.make_async_copy(dst_h.at[pl.ds(off, CH)], dv, sems[1]).wait()
        if with_rel:
            pltpu.make_async_copy(et_h.at[pl.ds(off, CH)], ev, sems[2]).wait()

    bsems0 = (semi0, semi1, semi2)
    bsems1 = (semi3, semi4, semi5)

    def grp_b(g, carry2):
        rows = g * L + lane
        for h in range(CPH):
            acc = jnp.zeros((L,), jnp.float32)
            for d in range(DH):
                col = _splat(h * DH + d)
                qv = plsc.load_gather(q_t, [rows, col])
                kv = plsc.load_gather(g0, [rows, col])
                if with_rel:
                    kv = kv + plsc.load_gather(rel_t, [rows, col])
                acc = acc + qv * kv
            acc = acc * 0.25
            acc = jnp.where(acc >= 0.0, acc, 0.2 * acc)
            acc = jnp.exp(acc)
            plsc.store_scatter(ex0, [rows, _splat(h)], acc)
        return carry2

    def body_b(ch, sv, dv, ev):
        cpk = pltpu.async_copy(kc.at[sv], g0, semg0)
        cpq = pltpu.async_copy(qc.at[dv], q_t, semg1)
        if with_rel:
            cpr = pltpu.async_copy(relp_s.at[ev], rel_t, sem)
        cpk.wait()
        cpq.wait()
        if with_rel:
            cpr.wait()
        lax.fori_loop(0, CH // L, grp_b, 0)
        pltpu.sync_copy(ex0, den_s.at[dv], add=True)
        pltpu.sync_copy(ex0, exc.at[pl.ds(ebase + ch * CH, CH)])

    issue_bidx(0, src0, dst0, et_t, bsems0)

    def pair_b(j, carry):
        a = 2 * j
        b = a + 1
        wait_bidx(a, src0, dst0, et_t, bsems0)
        issue_bidx(b, src1, dst1, et1_t, bsems1)
        body_b(a, src0, dst0, et_t)
        wait_bidx(b, src1, dst1, et1_t, bsems1)

        @pl.when(j < NPAIR - 1)
        def _pf():
            issue_bidx(a + 2, src0, dst0, et_t, bsems0)

        body_b(b, src1, dst1, et1_t)
        return carry

    lax.fori_loop(0, NPAIR, pair_b, 0)
    plsc.subcore_barrier()

    def issue_idx(ci, sv, dv, ev, sems):
        off = ebase + ci * CH
        pltpu.async_copy(src_h.at[pl.ds(off, CH)], sv, sems[0])
        pltpu.async_copy(dst_h.at[pl.ds(off, CH)], dv, sems[1])
        pltpu.async_copy(exc.at[pl.ds(off, CH)], ev, sems[2])

    def wait_idx(ci, sv, dv, ev, sems):
        off = ebase + ci * CH
        pltpu.make_async_copy(src_h.at[pl.ds(off, CH)], sv, sems[0]).wait()
        pltpu.make_async_copy(dst_h.at[pl.ds(off, CH)], dv, sems[1]).wait()
        pltpu.make_async_copy(exc.at[pl.ds(off, CH)], ev, sems[2]).wait()

    def edge_mul(gb, eb):
        def edge_c(e, carry2):
            exr = eb[e, pl.ds(0, DH)]
            for h in range(CPH):
                exs = jnp.take(exr, _splat(h))
                fv = gb[e, pl.ds(h * DH, DH)]
                gb[e, pl.ds(h * DH, DH)] = fv * exs
            return carry2

        lax.fori_loop(0, CH, edge_c, 0, unroll=4)

    for hop in range(HOPS):
        for u in range(NU):
            rb = nbase + u * RU
            pltpu.sync_copy(z64_h.at[pl.ds(u * RU, RU)], ua_t)
            pltpu.sync_copy(ua_t, agg_s.at[pl.ds(rb, RU)])
        plsc.subcore_barrier()

        sems0 = (semi0, semi1, semi2)
        sems1 = (semi3, semi4, semi5)
        issue_idx(0, src0, dst0, ex0, sems0)
        wait_idx(0, src0, dst0, ex0, sems0)
        pltpu.async_copy(feat_s.at[src0], g0, semg0)
        issue_idx(1, src1, dst1, ex1, sems1)

        def pair_c(j, carry):
            a = 2 * j
            b = a + 1
            wait_idx(b, src1, dst1, ex1, sems1)
            pltpu.async_copy(feat_s.at[src1], g1, semg1)
            pltpu.make_async_copy(feat_s.at[src0], g0, semg0).wait()
            edge_mul(g0, ex0)
            pltpu.sync_copy(g0, agg_s.at[dst0], add=True)

            @pl.when(j < NPAIR - 1)
            def _next_a():
                issue_idx(a + 2, src0, dst0, ex0, sems0)
                wait_idx(a + 2, src0, dst0, ex0, sems0)
                pltpu.async_copy(feat_s.at[src0], g0, semg0)

            pltpu.make_async_copy(feat_s.at[src1], g1, semg1).wait()
            edge_mul(g1, ex1)
            pltpu.sync_copy(g1, agg_s.at[dst1], add=True)

            @pl.when(j < NPAIR - 1)
            def _next_b():
                issue_idx(b + 2, src1, dst1, ex1, sems1)

            return carry

        lax.fori_loop(0, NPAIR, pair_c, 0)
        plsc.subcore_barrier()

        for u in range(NU):
            rb = nbase + u * RU
            cpa = pltpu.async_copy(agg_s.at[pl.ds(rb, RU)], ua_t, sem)
            cpd = pltpu.async_copy(den_s.at[pl.ds(rb, RU)], ud_t, semi4)
            cpv = pltpu.async_copy(vc.at[pl.ds(rb, RU)], uv_t, semi5)
            cpa.wait()
            cpd.wait()
            cpv.wait()

            def node_u(r, carry):
                dvec = ud_t[r, pl.ds(0, DH)]
                rcpv = (1.0 - ALPHA) / (dvec + 1e-16)
                for h in range(CPH):
                    rhv = jnp.take(rcpv, _splat(h))
                    av = ua_t[r, pl.ds(h * DH, DH)]
                    vv = uv_t[r, pl.ds(h * DH, DH)]
                    fnew = rhv * av + ALPHA * vv
                    ua_t[r, pl.ds(h * DH, DH)] = fnew
                return carry

            lax.fori_loop(0, RU, node_u, 0, unroll=2)
            pltpu.sync_copy(ua_t, feat_s.at[pl.ds(rb, RU)])
            if hop == HOPS - 1:
                pltpu.sync_copy(ua_t, fo.at[pl.ds(rb, RU)])
        plsc.subcore_barrier()


def _sc_layer(src, dst, etype, q2, k2, v2, relp2, with_rel):
    mesh = plsc.VectorSubcoreMesh(core_axis_name="c", subcore_axis_name="s")
    z64 = jnp.zeros((NPT, 64), jnp.float32)
    z16 = jnp.zeros((NPT, 16), jnp.float32)
    if not with_rel:
        etype = jnp.zeros((8,), jnp.int32)
        relp2 = jnp.zeros((2, 8, 64), jnp.float32)

    kern = pl.kernel(
        functools.partial(_sc_layer_body, with_rel),
        out_type=(
            jax.ShapeDtypeStruct((2, NPAD, 64), jnp.float32),
            jax.ShapeDtypeStruct((2, N_EDGES, 16), jnp.float32),
        ),
        mesh=mesh,
        compiler_params=pltpu.CompilerParams(needs_layout_passes=False, use_tc_tiling_on_sc=False),
        scratch_types=[
            pltpu.VMEM_SHARED((NPAD, 64), jnp.float32),
            pltpu.VMEM_SHARED((NPAD, 64), jnp.float32),
            pltpu.VMEM_SHARED((NPAD, 16), jnp.float32),
            pltpu.VMEM_SHARED((NUM_REL, 64), jnp.float32),
            pltpu.VMEM((CH, 64), jnp.float32),
            pltpu.VMEM((CH, 64), jnp.float32),
            pltpu.VMEM((CH, 16), jnp.float32),
            pltpu.VMEM((CH, 16), jnp.float32),
            pltpu.VMEM((CH,), jnp.int32),
            pltpu.VMEM((CH,), jnp.int32),
            pltpu.VMEM((CH,), jnp.int32),
            pltpu.VMEM((CH,), jnp.int32),
            pltpu.VMEM((CH, 64), jnp.float32),
            pltpu.VMEM((CH, 64), jnp.float32),
            pltpu.VMEM((CH,), jnp.int32),
            pltpu.VMEM((CH,), jnp.int32),
            pltpu.VMEM((RU, 64), jnp.float32),
            pltpu.VMEM((RU, 16), jnp.float32),
            pltpu.VMEM((RU, 64), jnp.float32),
            pltpu.SemaphoreType.DMA,
            pltpu.SemaphoreType.DMA,
            pltpu.SemaphoreType.DMA,
            pltpu.SemaphoreType.DMA,
            pltpu.SemaphoreType.DMA,
            pltpu.SemaphoreType.DMA,
            pltpu.SemaphoreType.DMA,
            pltpu.SemaphoreType.DMA,
            pltpu.SemaphoreType.DMA,
        ],
    )
    feat, _ex = kern(src, dst, etype, q2, k2, v2, relp2, z64, z16)
    return feat


def _split_heads(x, pad_to=None):
    m = x.shape[0]
    out = jnp.swapaxes(x.reshape(m, 2, 64), 0, 1)
    if pad_to is not None and pad_to > m:
        out = jnp.pad(out, ((0, 0), (0, pad_to - m), (0, 0)))
    return out


def _tc_proj3_body(x_ref, wq_ref, wk_ref, wv_ref, q_ref, k_ref, v_ref):
    x = x_ref[...]
    q_ref[...] = jnp.dot(x, wq_ref[...], preferred_element_type=jnp.float32)
    k_ref[...] = jnp.dot(x, wk_ref[...], preferred_element_type=jnp.float32)
    v_ref[...] = jnp.dot(x, wv_ref[...], preferred_element_type=jnp.float32)


def _tc_proj3(x, wq, wk, wv, bm):
    m = x.shape[0]
    spec_x = pl.BlockSpec((bm, D), lambda i: (i, 0))
    spec_w = pl.BlockSpec((D, D), lambda i: (0, 0))
    spec_o = pl.BlockSpec((bm, D), lambda i: (i, 0))
    shp = jax.ShapeDtypeStruct((m, D), jnp.float32)
    return pl.pallas_call(
        _tc_proj3_body,
        grid=(m // bm,),
        in_specs=[spec_x, spec_w, spec_w, spec_w],
        out_specs=[spec_o, spec_o, spec_o],
        out_shape=[shp, shp, shp],
    )(x, wq, wk, wv)


def _tc_proj1_body(x_ref, w_ref, o_ref):
    o_ref[...] = jnp.dot(x_ref[...], w_ref[...], preferred_element_type=jnp.float32)


def _tc_proj1(x, w):
    m = x.shape[0]
    return pl.pallas_call(
        _tc_proj1_body,
        out_shape=jax.ShapeDtypeStruct((m, D), jnp.float32),
    )(x, w)


def _elu(x):
    return jnp.where(x > 0.0, x, jnp.exp(x) - 1.0)


def _tc_res3_body(f_ref, h_ref, wq_ref, wk_ref, wv_ref, h1_ref, q_ref, k_ref, v_ref):
    h1 = _elu(f_ref[...] + h_ref[...])
    h1_ref[...] = h1
    q_ref[...] = jnp.dot(h1, wq_ref[...], preferred_element_type=jnp.float32)
    k_ref[...] = jnp.dot(h1, wk_ref[...], preferred_element_type=jnp.float32)
    v_ref[...] = jnp.dot(h1, wv_ref[...], preferred_element_type=jnp.float32)


def _tc_res3(f, h, wq, wk, wv, bm):
    m = f.shape[0]
    spec = pl.BlockSpec((bm, D), lambda i: (i, 0))
    spec_w = pl.BlockSpec((D, D), lambda i: (0, 0))
    shp = jax.ShapeDtypeStruct((m, D), jnp.float32)
    return pl.pallas_call(
        _tc_res3_body,
        grid=(m // bm,),
        in_specs=[spec, spec, spec_w, spec_w, spec_w],
        out_specs=[spec, spec, spec, spec],
        out_shape=[shp, shp, shp, shp],
    )(f, h, wq, wk, wv)


def _tc_res_body(f_ref, h_ref, o_ref):
    o_ref[...] = _elu(f_ref[...] + h_ref[...])


def _tc_res(f, h, bm):
    m = f.shape[0]
    spec = pl.BlockSpec((bm, D), lambda i: (i, 0))
    return pl.pallas_call(
        _tc_res_body,
        grid=(m // bm,),
        in_specs=[spec, spec],
        out_specs=spec,
        out_shape=jax.ShapeDtypeStruct((m, D), jnp.float32),
    )(f, h)


def kernel(edge_index, edge_type, ent_table, rel_table, Wq1, Wk1, Wv1, Wr1, Wq2, Wk2, Wv2):
    src = edge_index[0]
    dst = edge_index[1]

    q1, k1, v1 = _tc_proj3(ent_table, Wq1, Wk1, Wv1, bm=1000)
    relp = _tc_proj1(rel_table, Wr1)

    feat1 = _sc_layer(src, dst, edge_type,
                      _split_heads(q1, NPAD), _split_heads(k1, NPAD),
                      _split_heads(v1, NPAD),
                      _split_heads(relp), with_rel=True)
    feat1 = jnp.swapaxes(feat1[:, :N_NODES], 0, 1).reshape(N_NODES, D)

    h1, q2, k2, v2 = _tc_res3(feat1, ent_table, Wq2, Wk2, Wv2, bm=1000)

    feat2 = _sc_layer(src, dst, None,
                      _split_heads(q2, NPAD), _split_heads(k2, NPAD),
                      _split_heads(v2, NPAD),
                      None, with_rel=False)
    feat2 = jnp.swapaxes(feat2[:, :N_NODES], 0, 1).reshape(N_NODES, D)

    return _tc_res(feat2, h1, bm=1000)


# speedup vs baseline: 1.6013x; 1.6013x over previous
import functools

import jax
import jax.numpy as jnp
from jax import lax
from jax.experimental import pallas as pl
from jax.experimental.pallas import tpu as pltpu
from jax.experimental.pallas import tpu_sc as plsc

N_NODES = 10000
N_EDGES = 320000
NUM_REL = 256
D = 128
H = 8
DH = 16
HOPS = 3
ALPHA = 0.15

NC = 2
NS = 16
L = 16

CH = 80
EPT = N_EDGES // NS
NCHUNK = EPT // CH
NPAD = 10240
NPT = NPAD // NS
RU = 64
NU = NPT // RU
CPH = 4
NPAIR = NCHUNK // 2


def _lane_iota():
    return lax.iota(jnp.int32, L)


def _splat(x):
    return jnp.full((L,), x, jnp.int32)


def _sc_layer_body(with_rel, src_h, dst_h, et_h, q_h, k_h, v_h, relp_h, z64_h,
                   z16_h, feat_o, ex_o,
                   feat_s, agg_s, den_s, relp_s,
                   g0, g1, ex0, ex1, src0, src1, dst0, dst1,
                   q_t, rel_t, et_t, et1_t, ua_t, ud_t, uv_t,
                   semi0, semi1, semg0, semg1, sem, semi2, semi3, semi4, semi5, sems0d, sems1d):
    c = lax.axis_index("c")
    s = lax.axis_index("s")
    ebase = s * EPT
    nbase = s * NPT
    lane = _lane_iota()

    qc = q_h.at[c]
    kc = k_h.at[c]
    vc = v_h.at[c]
    fo = feat_o.at[c]
    exc = ex_o.at[c]

    for u in range(NU):
        rb = nbase + u * RU
        pltpu.sync_copy(vc.at[pl.ds(rb, RU)], uv_t)
        pltpu.sync_copy(uv_t, feat_s.at[pl.ds(rb, RU)])
        pltpu.sync_copy(z16_h.at[pl.ds(u * RU, RU)], ud_t)
        pltpu.sync_copy(ud_t, den_s.at[pl.ds(rb, RU)])
    if with_rel:
        @pl.when(s == 0)
        def _copy_relp():
            for j in range(NUM_REL // RU):
                pltpu.sync_copy(relp_h.at[c, pl.ds(j * RU, RU)], ua_t)
                pltpu.sync_copy(ua_t, relp_s.at[pl.ds(j * RU, RU)])
    plsc.subcore_barrier()

    def issue_bidx(ci, sv, dv, ev, sems):
        off = ebase + ci * CH
        pltpu.async_copy(src_h.at[pl.ds(off, CH)], sv, sems[0])
        pltpu.async_copy(dst_h.at[pl.ds(off, CH)], dv, sems[1])
        if with_rel:
            pltpu.async_copy(et_h.at[pl.ds(off, CH)], ev, sems[2])

    def wait_bidx(ci, sv, dv, ev, sems):
        off = ebase + ci * CH
        pltpu.make_async_copy(src_h.at[pl.ds(off, CH)], sv, sems[0]).wait()
        pltpu.make_async_copy(dst_h.at[pl.ds(off, CH)], dv, sems[1]).wait()
        if with_rel:
            pltpu.make_async_copy(et_h.at[pl.ds(off, CH)], ev, sems[2]).wait()

    bsems0 = (semi0, semi1, semi2)
    bsems1 = (semi3, semi4, semi5)

    def edge_b(e, carry2):
        row = jnp.zeros((L,), jnp.float32)
        for h in range(CPH):
            kv = g0[e, pl.ds(h * DH, DH)]
            qv = q_t[e, pl.ds(h * DH, DH)]
            if with_rel:
                rv = rel_t[e, pl.ds(h * DH, DH)]
                kv = kv + rv
            sh = jnp.sum(qv * kv)
            row = row + jnp.where(lane == h, sh, 0.0)
        row = row * 0.25
        row = jnp.where(row >= 0.0, row, 0.2 * row)
        exv = jnp.exp(row)
        ex0[e, pl.ds(0, DH)] = exv
        return carry2

    def body_b(ch, sv, dv, ev):
        cpk = pltpu.async_copy(kc.at[sv], g0, semg0)
        cpq = pltpu.async_copy(qc.at[dv], q_t, semg1)
        if with_rel:
            cpr = pltpu.async_copy(relp_s.at[ev], rel_t, sem)
        cpk.wait()
        cpq.wait()
        if with_rel:
            cpr.wait()
        lax.fori_loop(0, CH, edge_b, 0, unroll=2)
        pltpu.sync_copy(ex0, den_s.at[dv], add=True)
        pltpu.sync_copy(ex0, exc.at[pl.ds(ebase + ch * CH, CH)])

    issue_bidx(0, src0, dst0, et_t, bsems0)

    def pair_b(j, carry):
        a = 2 * j
        b = a + 1
        wait_bidx(a, src0, dst0, et_t, bsems0)
        issue_bidx(b, src1, dst1, et1_t, bsems1)
        body_b(a, src0, dst0, et_t)
        wait_bidx(b, src1, dst1, et1_t, bsems1)

        @pl.when(j < NPAIR - 1)
        def _pf():
            issue_bidx(a + 2, src0, dst0, et_t, bsems0)

        body_b(b, src1, dst1, et1_t)
        return carry

    lax.fori_loop(0, NPAIR, pair_b, 0)
    plsc.subcore_barrier()

    def issue_idx(ci, sv, dv, ev, sems):
        off = ebase + ci * CH
        pltpu.async_copy(src_h.at[pl.ds(off, CH)], sv, sems[0])
        pltpu.async_copy(dst_h.at[pl.ds(off, CH)], dv, sems[1])
        pltpu.async_copy(exc.at[pl.ds(off, CH)], ev, sems[2])

    def wait_idx(ci, sv, dv, ev, sems):
        off = ebase + ci * CH
        pltpu.make_async_copy(src_h.at[pl.ds(off, CH)], sv, sems[0]).wait()
        pltpu.make_async_copy(dst_h.at[pl.ds(off, CH)], dv, sems[1]).wait()
        pltpu.make_async_copy(exc.at[pl.ds(off, CH)], ev, sems[2]).wait()

    def edge_mul(gb, eb):
        def edge_c(e, carry2):
            exr = eb[e, pl.ds(0, DH)]
            for h in range(CPH):
                exs = jnp.take(exr, _splat(h))
                fv = gb[e, pl.ds(h * DH, DH)]
                gb[e, pl.ds(h * DH, DH)] = fv * exs
            return carry2

        lax.fori_loop(0, CH, edge_c, 0, unroll=8)

    for hop in range(HOPS):
        for u in range(NU):
            rb = nbase + u * RU
            pltpu.sync_copy(z64_h.at[pl.ds(u * RU, RU)], ua_t)
            pltpu.sync_copy(ua_t, agg_s.at[pl.ds(rb, RU)])
        plsc.subcore_barrier()

        sems0 = (semi0, semi1, semi2)
        sems1 = (semi3, semi4, semi5)
        issue_idx(0, src0, dst0, ex0, sems0)
        wait_idx(0, src0, dst0, ex0, sems0)
        pltpu.async_copy(feat_s.at[src0], g0, semg0)
        issue_idx(1, src1, dst1, ex1, sems1)

        def pair_c(j, carry):
            a = 2 * j
            b = a + 1
            wait_idx(b, src1, dst1, ex1, sems1)
            pltpu.async_copy(feat_s.at[src1], g1, semg1)
            pltpu.make_async_copy(feat_s.at[src0], g0, semg0).wait()
            edge_mul(g0, ex0)
            pltpu.async_copy(g0, agg_s.at[dst0], sems0d, add=True)

            @pl.when(j < NPAIR - 1)
            def _next_a():
                issue_idx(a + 2, src0, dst0, ex0, sems0)
                pltpu.make_async_copy(g0, agg_s.at[dst0], sems0d).wait()
                wait_idx(a + 2, src0, dst0, ex0, sems0)
                pltpu.async_copy(feat_s.at[src0], g0, semg0)

            pltpu.make_async_copy(feat_s.at[src1], g1, semg1).wait()
            edge_mul(g1, ex1)
            pltpu.async_copy(g1, agg_s.at[dst1], sems1d, add=True)

            @pl.when(j < NPAIR - 1)
            def _next_b():
                issue_idx(b + 2, src1, dst1, ex1, sems1)
                pltpu.make_async_copy(g1, agg_s.at[dst1], sems1d).wait()

            return carry

        lax.fori_loop(0, NPAIR, pair_c, 0)
        pltpu.make_async_copy(g0, agg_s.at[dst0], sems0d).wait()
        pltpu.make_async_copy(g1, agg_s.at[dst1], sems1d).wait()
        plsc.subcore_barrier()

        for u in range(NU):
            rb = nbase + u * RU
            cpa = pltpu.async_copy(agg_s.at[pl.ds(rb, RU)], ua_t, sem)
            cpd = pltpu.async_copy(den_s.at[pl.ds(rb, RU)], ud_t, semi4)
            cpv = pltpu.async_copy(vc.at[pl.ds(rb, RU)], uv_t, semi5)
            cpa.wait()
            cpd.wait()
            cpv.wait()

            def node_u(r, carry):
                dvec = ud_t[r, pl.ds(0, DH)]
                rcpv = (1.0 - ALPHA) / (dvec + 1e-16)
                for h in range(CPH):
                    rhv = jnp.take(rcpv, _splat(h))
                    av = ua_t[r, pl.ds(h * DH, DH)]
                    vv = uv_t[r, pl.ds(h * DH, DH)]
                    fnew = rhv * av + ALPHA * vv
                    ua_t[r, pl.ds(h * DH, DH)] = fnew
                return carry

            lax.fori_loop(0, RU, node_u, 0, unroll=2)
            pltpu.sync_copy(ua_t, feat_s.at[pl.ds(rb, RU)])
            if hop == HOPS - 1:
                pltpu.sync_copy(ua_t, fo.at[pl.ds(rb, RU)])
        plsc.subcore_barrier()


def _sc_layer(src, dst, etype, q2, k2, v2, relp2, with_rel):
    mesh = plsc.VectorSubcoreMesh(core_axis_name="c", subcore_axis_name="s")
    z64 = jnp.zeros((NPT, 64), jnp.float32)
    z16 = jnp.zeros((NPT, 16), jnp.float32)
    if not with_rel:
        etype = jnp.zeros((8,), jnp.int32)
        relp2 = jnp.zeros((2, 8, 64), jnp.float32)

    kern = pl.kernel(
        functools.partial(_sc_layer_body, with_rel),
        out_type=(
            jax.ShapeDtypeStruct((2, NPAD, 64), jnp.float32),
            jax.ShapeDtypeStruct((2, N_EDGES, 16), jnp.float32),
        ),
        mesh=mesh,
        compiler_params=pltpu.CompilerParams(needs_layout_passes=False, use_tc_tiling_on_sc=False),
        scratch_types=[
            pltpu.VMEM_SHARED((NPAD, 64), jnp.float32),
            pltpu.VMEM_SHARED((NPAD, 64), jnp.float32),
            pltpu.VMEM_SHARED((NPAD, 16), jnp.float32),
            pltpu.VMEM_SHARED((NUM_REL, 64), jnp.float32),
            pltpu.VMEM((CH, 64), jnp.float32),
            pltpu.VMEM((CH, 64), jnp.float32),
            pltpu.VMEM((CH, 16), jnp.float32),
            pltpu.VMEM((CH, 16), jnp.float32),
            pltpu.VMEM((CH,), jnp.int32),
            pltpu.VMEM((CH,), jnp.int32),
            pltpu.VMEM((CH,), jnp.int32),
            pltpu.VMEM((CH,), jnp.int32),
            pltpu.VMEM((CH, 64), jnp.float32),
            pltpu.VMEM((CH, 64), jnp.float32),
            pltpu.VMEM((CH,), jnp.int32),
            pltpu.VMEM((CH,), jnp.int32),
            pltpu.VMEM((RU, 64), jnp.float32),
            pltpu.VMEM((RU, 16), jnp.float32),
            pltpu.VMEM((RU, 64), jnp.float32),
            pltpu.SemaphoreType.DMA,
            pltpu.SemaphoreType.DMA,
            pltpu.SemaphoreType.DMA,
            pltpu.SemaphoreType.DMA,
            pltpu.SemaphoreType.DMA,
            pltpu.SemaphoreType.DMA,
            pltpu.SemaphoreType.DMA,
            pltpu.SemaphoreType.DMA,
            pltpu.SemaphoreType.DMA,
            pltpu.SemaphoreType.DMA,
            pltpu.SemaphoreType.DMA,
        ],
    )
    feat, _ex = kern(src, dst, etype, q2, k2, v2, relp2, z64, z16)
    return feat


def _split_heads(x, pad_to=None):
    m = x.shape[0]
    out = jnp.swapaxes(x.reshape(m, 2, 64), 0, 1)
    if pad_to is not None and pad_to > m:
        out = jnp.pad(out, ((0, 0), (0, pad_to - m), (0, 0)))
    return out


def _tc_proj3_body(x_ref, wq_ref, wk_ref, wv_ref, q_ref, k_ref, v_ref):
    x = x_ref[...]
    q_ref[...] = jnp.dot(x, wq_ref[...], preferred_element_type=jnp.float32)
    k_ref[...] = jnp.dot(x, wk_ref[...], preferred_element_type=jnp.float32)
    v_ref[...] = jnp.dot(x, wv_ref[...], preferred_element_type=jnp.float32)


def _tc_proj3(x, wq, wk, wv, bm):
    m = x.shape[0]
    spec_x = pl.BlockSpec((bm, D), lambda i: (i, 0))
    spec_w = pl.BlockSpec((D, D), lambda i: (0, 0))
    spec_o = pl.BlockSpec((bm, D), lambda i: (i, 0))
    shp = jax.ShapeDtypeStruct((m, D), jnp.float32)
    return pl.pallas_call(
        _tc_proj3_body,
        grid=(m // bm,),
        in_specs=[spec_x, spec_w, spec_w, spec_w],
        out_specs=[spec_o, spec_o, spec_o],
        out_shape=[shp, shp, shp],
    )(x, wq, wk, wv)


def _tc_proj1_body(x_ref, w_ref, o_ref):
    o_ref[...] = jnp.dot(x_ref[...], w_ref[...], preferred_element_type=jnp.float32)


def _tc_proj1(x, w):
    m = x.shape[0]
    return pl.pallas_call(
        _tc_proj1_body,
        out_shape=jax.ShapeDtypeStruct((m, D), jnp.float32),
    )(x, w)


def _elu(x):
    return jnp.where(x > 0.0, x, jnp.exp(x) - 1.0)


def _tc_res3_body(f_ref, h_ref, wq_ref, wk_ref, wv_ref, h1_ref, q_ref, k_ref, v_ref):
    h1 = _elu(f_ref[...] + h_ref[...])
    h1_ref[...] = h1
    q_ref[...] = jnp.dot(h1, wq_ref[...], preferred_element_type=jnp.float32)
    k_ref[...] = jnp.dot(h1, wk_ref[...], preferred_element_type=jnp.float32)
    v_ref[...] = jnp.dot(h1, wv_ref[...], preferred_element_type=jnp.float32)


def _tc_res3(f, h, wq, wk, wv, bm):
    m = f.shape[0]
    spec = pl.BlockSpec((bm, D), lambda i: (i, 0))
    spec_w = pl.BlockSpec((D, D), lambda i: (0, 0))
    shp = jax.ShapeDtypeStruct((m, D), jnp.float32)
    return pl.pallas_call(
        _tc_res3_body,
        grid=(m // bm,),
        in_specs=[spec, spec, spec_w, spec_w, spec_w],
        out_specs=[spec, spec, spec, spec],
        out_shape=[shp, shp, shp, shp],
    )(f, h, wq, wk, wv)


def _tc_res_body(f_ref, h_ref, o_ref):
    o_ref[...] = _elu(f_ref[...] + h_ref[...])


def _tc_res(f, h, bm):
    m = f.shape[0]
    spec = pl.BlockSpec((bm, D), lambda i: (i, 0))
    return pl.pallas_call(
        _tc_res_body,
        grid=(m // bm,),
        in_specs=[spec, spec],
        out_specs=spec,
        out_shape=jax.ShapeDtypeStruct((m, D), jnp.float32),
    )(f, h)


def kernel(edge_index, edge_type, ent_table, rel_table, Wq1, Wk1, Wv1, Wr1, Wq2, Wk2, Wv2):
    src = edge_index[0]
    dst = edge_index[1]

    q1, k1, v1 = _tc_proj3(ent_table, Wq1, Wk1, Wv1, bm=1000)
    relp = _tc_proj1(rel_table, Wr1)

    feat1 = _sc_layer(src, dst, edge_type,
                      _split_heads(q1, NPAD), _split_heads(k1, NPAD),
                      _split_heads(v1, NPAD),
                      _split_heads(relp), with_rel=True)
    feat1 = jnp.swapaxes(feat1[:, :N_NODES], 0, 1).reshape(N_NODES, D)

    h1, q2, k2, v2 = _tc_res3(feat1, ent_table, Wq2, Wk2, Wv2, bm=1000)

    feat2 = _sc_layer(src, dst, None,
                      _split_heads(q2, NPAD), _split_heads(k2, NPAD),
                      _split_heads(v2, NPAD),
                      None, with_rel=False)
    feat2 = jnp.swapaxes(feat2[:, :N_NODES], 0, 1).reshape(N_NODES, D)

    return _tc_res(feat2, h1, bm=1000)


# speedup vs baseline: 1.6543x; 1.0331x over previous
import functools

import jax
import jax.numpy as jnp
from jax import lax
from jax.experimental import pallas as pl
from jax.experimental.pallas import tpu as pltpu
from jax.experimental.pallas import tpu_sc as plsc

N_NODES = 10000
N_EDGES = 320000
NUM_REL = 256
D = 128
H = 8
DH = 16
HOPS = 3
ALPHA = 0.15

NC = 2
NS = 16
L = 16

CH = 80
EPT = N_EDGES // NS
NCHUNK = EPT // CH
NPAD = 10240
NPT = NPAD // NS
RU = 64
NU = NPT // RU
CPH = 4
NPAIR = NCHUNK // 2


def _lane_iota():
    return lax.iota(jnp.int32, L)


def _splat(x):
    return jnp.full((L,), x, jnp.int32)


def _sc_layer_body(with_rel, src_h, dst_h, et_h, q_h, k_h, v_h, relp_h, z64_h,
                   z16_h, feat_o, ex_o,
                   feat_s, agg_s, den_s, relp_s,
                   g0, g1, ex0, ex1, src0, src1, dst0, dst1,
                   q_t, rel_t, et_t, et1_t, ua_t, ud_t, uv_t,
                   semi0, semi1, semg0, semg1, sem, semi2, semi3, semi4, semi5, sems0d, sems1d, sembd, sembe):
    c = lax.axis_index("c")
    s = lax.axis_index("s")
    ebase = s * EPT
    nbase = s * NPT
    lane = _lane_iota()

    qc = q_h.at[c]
    kc = k_h.at[c]
    vc = v_h.at[c]
    fo = feat_o.at[c]
    exc = ex_o.at[c]

    for u in range(NU):
        rb = nbase + u * RU
        pltpu.sync_copy(vc.at[pl.ds(rb, RU)], uv_t)
        pltpu.sync_copy(uv_t, feat_s.at[pl.ds(rb, RU)])
        pltpu.sync_copy(z16_h.at[pl.ds(u * RU, RU)], ud_t)
        pltpu.sync_copy(ud_t, den_s.at[pl.ds(rb, RU)])
    if with_rel:
        @pl.when(s == 0)
        def _copy_relp():
            for j in range(NUM_REL // RU):
                pltpu.sync_copy(relp_h.at[c, pl.ds(j * RU, RU)], ua_t)
                pltpu.sync_copy(ua_t, relp_s.at[pl.ds(j * RU, RU)])
    plsc.subcore_barrier()

    def issue_bidx(ci, sv, dv, ev, sems):
        off = ebase + ci * CH
        pltpu.async_copy(src_h.at[pl.ds(off, CH)], sv, sems[0])
        pltpu.async_copy(dst_h.at[pl.ds(off, CH)], dv, sems[1])
        if with_rel:
            pltpu.async_copy(et_h.at[pl.ds(off, CH)], ev, sems[2])

    def wait_bidx(ci, sv, dv, ev, sems):
        off = ebase + ci * CH
        pltpu.make_async_copy(src_h.at[pl.ds(off, CH)], sv, sems[0]).wait()
        pltpu.make_async_copy(dst_h.at[pl.ds(off, CH)], dv, sems[1]).wait()
        if with_rel:
            pltpu.make_async_copy(et_h.at[pl.ds(off, CH)], ev, sems[2]).wait()

    bsems0 = (semi0, semi1, semi2)
    bsems1 = (semi3, semi4, semi5)

    def edge_b(e, carry2):
        row = jnp.zeros((L,), jnp.float32)
        for h in range(CPH):
            kv = g0[e, pl.ds(h * DH, DH)]
            qv = q_t[e, pl.ds(h * DH, DH)]
            if with_rel:
                rv = rel_t[e, pl.ds(h * DH, DH)]
                kv = kv + rv
            sh = jnp.sum(qv * kv)
            row = row + jnp.where(lane == h, sh, 0.0)
        row = row * 0.25
        row = jnp.where(row >= 0.0, row, 0.2 * row)
        exv = jnp.exp(row)
        ex0[e, pl.ds(0, DH)] = exv
        return carry2

    def body_b(ch, sv, dv, ev):
        cpk = pltpu.async_copy(kc.at[sv], g0, semg0)
        cpq = pltpu.async_copy(qc.at[dv], q_t, semg1)
        if with_rel:
            cpr = pltpu.async_copy(relp_s.at[ev], rel_t, sem)

        @pl.when(ch > 0)
        def _drain_prev():
            pltpu.make_async_copy(ex0, den_s.at[dv], sembd).wait()
            pltpu.make_async_copy(ex0, exc.at[pl.ds(ebase, CH)], sembe).wait()

        cpk.wait()
        cpq.wait()
        if with_rel:
            cpr.wait()
        lax.fori_loop(0, CH, edge_b, 0, unroll=4)
        pltpu.async_copy(ex0, den_s.at[dv], sembd, add=True)
        pltpu.async_copy(ex0, exc.at[pl.ds(ebase + ch * CH, CH)], sembe)

    issue_bidx(0, src0, dst0, et_t, bsems0)

    def pair_b(j, carry):
        a = 2 * j
        b = a + 1
        wait_bidx(a, src0, dst0, et_t, bsems0)
        issue_bidx(b, src1, dst1, et1_t, bsems1)
        body_b(a, src0, dst0, et_t)
        wait_bidx(b, src1, dst1, et1_t, bsems1)

        @pl.when(j < NPAIR - 1)
        def _pf():
            issue_bidx(a + 2, src0, dst0, et_t, bsems0)

        body_b(b, src1, dst1, et1_t)
        return carry

    lax.fori_loop(0, NPAIR, pair_b, 0)
    pltpu.make_async_copy(ex0, den_s.at[dst1], sembd).wait()
    pltpu.make_async_copy(ex0, exc.at[pl.ds(ebase, CH)], sembe).wait()
    plsc.subcore_barrier()

    def issue_idx(ci, sv, dv, ev, sems):
        off = ebase + ci * CH
        pltpu.async_copy(src_h.at[pl.ds(off, CH)], sv, sems[0])
        pltpu.async_copy(dst_h.at[pl.ds(off, CH)], dv, sems[1])
        pltpu.async_copy(exc.at[pl.ds(off, CH)], ev, sems[2])

    def wait_idx(ci, sv, dv, ev, sems):
        off = ebase + ci * CH
        pltpu.make_async_copy(src_h.at[pl.ds(off, CH)], sv, sems[0]).wait()
        pltpu.make_async_copy(dst_h.at[pl.ds(off, CH)], dv, sems[1]).wait()
        pltpu.make_async_copy(exc.at[pl.ds(off, CH)], ev, sems[2]).wait()

    def edge_mul(gb, eb):
        def edge_c(e, carry2):
            exr = eb[e, pl.ds(0, DH)]
            for h in range(CPH):
                exs = jnp.take(exr, _splat(h))
                fv = gb[e, pl.ds(h * DH, DH)]
                gb[e, pl.ds(h * DH, DH)] = fv * exs
            return carry2

        lax.fori_loop(0, CH, edge_c, 0, unroll=8)

    for hop in range(HOPS):
        for u in range(NU):
            rb = nbase + u * RU
            pltpu.sync_copy(z64_h.at[pl.ds(u * RU, RU)], ua_t)
            pltpu.sync_copy(ua_t, agg_s.at[pl.ds(rb, RU)])
        plsc.subcore_barrier()

        sems0 = (semi0, semi1, semi2)
        sems1 = (semi3, semi4, semi5)
        issue_idx(0, src0, dst0, ex0, sems0)
        wait_idx(0, src0, dst0, ex0, sems0)
        pltpu.async_copy(feat_s.at[src0], g0, semg0)
        issue_idx(1, src1, dst1, ex1, sems1)

        def pair_c(j, carry):
            a = 2 * j
            b = a + 1
            wait_idx(b, src1, dst1, ex1, sems1)
            pltpu.async_copy(feat_s.at[src1], g1, semg1)
            pltpu.make_async_copy(feat_s.at[src0], g0, semg0).wait()
            edge_mul(g0, ex0)
            pltpu.async_copy(g0, agg_s.at[dst0], sems0d, add=True)

            @pl.when(j < NPAIR - 1)
            def _next_a():
                issue_idx(a + 2, src0, dst0, ex0, sems0)
                pltpu.make_async_copy(g0, agg_s.at[dst0], sems0d).wait()
                wait_idx(a + 2, src0, dst0, ex0, sems0)
                pltpu.async_copy(feat_s.at[src0], g0, semg0)

            pltpu.make_async_copy(feat_s.at[src1], g1, semg1).wait()
            edge_mul(g1, ex1)
            pltpu.async_copy(g1, agg_s.at[dst1], sems1d, add=True)

            @pl.when(j < NPAIR - 1)
            def _next_b():
                issue_idx(b + 2, src1, dst1, ex1, sems1)
                pltpu.make_async_copy(g1, agg_s.at[dst1], sems1d).wait()

            return carry

        lax.fori_loop(0, NPAIR, pair_c, 0)
        pltpu.make_async_copy(g0, agg_s.at[dst0], sems0d).wait()
        pltpu.make_async_copy(g1, agg_s.at[dst1], sems1d).wait()
        plsc.subcore_barrier()

        for u in range(NU):
            rb = nbase + u * RU
            cpa = pltpu.async_copy(agg_s.at[pl.ds(rb, RU)], ua_t, sem)
            cpd = pltpu.async_copy(den_s.at[pl.ds(rb, RU)], ud_t, semi4)
            cpv = pltpu.async_copy(vc.at[pl.ds(rb, RU)], uv_t, semi5)
            cpa.wait()
            cpd.wait()
            cpv.wait()

            def node_u(r, carry):
                dvec = ud_t[r, pl.ds(0, DH)]
                rcpv = (1.0 - ALPHA) / (dvec + 1e-16)
                for h in range(CPH):
                    rhv = jnp.take(rcpv, _splat(h))
                    av = ua_t[r, pl.ds(h * DH, DH)]
                    vv = uv_t[r, pl.ds(h * DH, DH)]
                    fnew = rhv * av + ALPHA * vv
                    ua_t[r, pl.ds(h * DH, DH)] = fnew
                return carry

            lax.fori_loop(0, RU, node_u, 0, unroll=2)
            pltpu.sync_copy(ua_t, feat_s.at[pl.ds(rb, RU)])
            if hop == HOPS - 1:
                pltpu.sync_copy(ua_t, fo.at[pl.ds(rb, RU)])
        plsc.subcore_barrier()


def _sc_layer(src, dst, etype, q2, k2, v2, relp2, with_rel):
    mesh = plsc.VectorSubcoreMesh(core_axis_name="c", subcore_axis_name="s")
    z64 = jnp.zeros((NPT, 64), jnp.float32)
    z16 = jnp.zeros((NPT, 16), jnp.float32)
    if not with_rel:
        etype = jnp.zeros((8,), jnp.int32)
        relp2 = jnp.zeros((2, 8, 64), jnp.float32)

    kern = pl.kernel(
        functools.partial(_sc_layer_body, with_rel),
        out_type=(
            jax.ShapeDtypeStruct((2, NPAD, 64), jnp.float32),
            jax.ShapeDtypeStruct((2, N_EDGES, 16), jnp.float32),
        ),
        mesh=mesh,
        compiler_params=pltpu.CompilerParams(needs_layout_passes=False, use_tc_tiling_on_sc=False),
        scratch_types=[
            pltpu.VMEM_SHARED((NPAD, 64), jnp.float32),
            pltpu.VMEM_SHARED((NPAD, 64), jnp.float32),
            pltpu.VMEM_SHARED((NPAD, 16), jnp.float32),
            pltpu.VMEM_SHARED((NUM_REL, 64), jnp.float32),
            pltpu.VMEM((CH, 64), jnp.float32),
            pltpu.VMEM((CH, 64), jnp.float32),
            pltpu.VMEM((CH, 16), jnp.float32),
            pltpu.VMEM((CH, 16), jnp.float32),
            pltpu.VMEM((CH,), jnp.int32),
            pltpu.VMEM((CH,), jnp.int32),
            pltpu.VMEM((CH,), jnp.int32),
            pltpu.VMEM((CH,), jnp.int32),
            pltpu.VMEM((CH, 64), jnp.float32),
            pltpu.VMEM((CH, 64), jnp.float32),
            pltpu.VMEM((CH,), jnp.int32),
            pltpu.VMEM((CH,), jnp.int32),
            pltpu.VMEM((RU, 64), jnp.float32),
            pltpu.VMEM((RU, 16), jnp.float32),
            pltpu.VMEM((RU, 64), jnp.float32),
            pltpu.SemaphoreType.DMA,
            pltpu.SemaphoreType.DMA,
            pltpu.SemaphoreType.DMA,
            pltpu.SemaphoreType.DMA,
            pltpu.SemaphoreType.DMA,
            pltpu.SemaphoreType.DMA,
            pltpu.SemaphoreType.DMA,
            pltpu.SemaphoreType.DMA,
            pltpu.SemaphoreType.DMA,
            pltpu.SemaphoreType.DMA,
            pltpu.SemaphoreType.DMA,
            pltpu.SemaphoreType.DMA,
            pltpu.SemaphoreType.DMA,
        ],
    )
    feat, _ex = kern(src, dst, etype, q2, k2, v2, relp2, z64, z16)
    return feat


def _split_heads(x, pad_to=None):
    m = x.shape[0]
    out = jnp.swapaxes(x.reshape(m, 2, 64), 0, 1)
    if pad_to is not None and pad_to > m:
        out = jnp.pad(out, ((0, 0), (0, pad_to - m), (0, 0)))
    return out


def _tc_proj3_body(x_ref, wq_ref, wk_ref, wv_ref, q_ref, k_ref, v_ref):
    x = x_ref[...]
    q_ref[...] = jnp.dot(x, wq_ref[...], preferred_element_type=jnp.float32)
    k_ref[...] = jnp.dot(x, wk_ref[...], preferred_element_type=jnp.float32)
    v_ref[...] = jnp.dot(x, wv_ref[...], preferred_element_type=jnp.float32)


def _tc_proj3(x, wq, wk, wv, bm):
    m = x.shape[0]
    spec_x = pl.BlockSpec((bm, D), lambda i: (i, 0))
    spec_w = pl.BlockSpec((D, D), lambda i: (0, 0))
    spec_o = pl.BlockSpec((bm, D), lambda i: (i, 0))
    shp = jax.ShapeDtypeStruct((m, D), jnp.float32)
    return pl.pallas_call(
        _tc_proj3_body,
        grid=(m // bm,),
        in_specs=[spec_x, spec_w, spec_w, spec_w],
        out_specs=[spec_o, spec_o, spec_o],
        out_shape=[shp, shp, shp],
    )(x, wq, wk, wv)


def _tc_proj1_body(x_ref, w_ref, o_ref):
    o_ref[...] = jnp.dot(x_ref[...], w_ref[...], preferred_element_type=jnp.float32)


def _tc_proj1(x, w):
    m = x.shape[0]
    return pl.pallas_call(
        _tc_proj1_body,
        out_shape=jax.ShapeDtypeStruct((m, D), jnp.float32),
    )(x, w)


def _elu(x):
    return jnp.where(x > 0.0, x, jnp.exp(x) - 1.0)


def _tc_res3_body(f_ref, h_ref, wq_ref, wk_ref, wv_ref, h1_ref, q_ref, k_ref, v_ref):
    h1 = _elu(f_ref[...] + h_ref[...])
    h1_ref[...] = h1
    q_ref[...] = jnp.dot(h1, wq_ref[...], preferred_element_type=jnp.float32)
    k_ref[...] = jnp.dot(h1, wk_ref[...], preferred_element_type=jnp.float32)
    v_ref[...] = jnp.dot(h1, wv_ref[...], preferred_element_type=jnp.float32)


def _tc_res3(f, h, wq, wk, wv, bm):
    m = f.shape[0]
    spec = pl.BlockSpec((bm, D), lambda i: (i, 0))
    spec_w = pl.BlockSpec((D, D), lambda i: (0, 0))
    shp = jax.ShapeDtypeStruct((m, D), jnp.float32)
    return pl.pallas_call(
        _tc_res3_body,
        grid=(m // bm,),
        in_specs=[spec, spec, spec_w, spec_w, spec_w],
        out_specs=[spec, spec, spec, spec],
        out_shape=[shp, shp, shp, shp],
    )(f, h, wq, wk, wv)


def _tc_res_body(f_ref, h_ref, o_ref):
    o_ref[...] = _elu(f_ref[...] + h_ref[...])


def _tc_res(f, h, bm):
    m = f.shape[0]
    spec = pl.BlockSpec((bm, D), lambda i: (i, 0))
    return pl.pallas_call(
        _tc_res_body,
        grid=(m // bm,),
        in_specs=[spec, spec],
        out_specs=spec,
        out_shape=jax.ShapeDtypeStruct((m, D), jnp.float32),
    )(f, h)


def kernel(edge_index, edge_type, ent_table, rel_table, Wq1, Wk1, Wv1, Wr1, Wq2, Wk2, Wv2):
    src = edge_index[0]
    dst = edge_index[1]

    q1, k1, v1 = _tc_proj3(ent_table, Wq1, Wk1, Wv1, bm=1000)
    relp = _tc_proj1(rel_table, Wr1)

    feat1 = _sc_layer(src, dst, edge_type,
                      _split_heads(q1, NPAD), _split_heads(k1, NPAD),
                      _split_heads(v1, NPAD),
                      _split_heads(relp), with_rel=True)
    feat1 = jnp.swapaxes(feat1[:, :N_NODES], 0, 1).reshape(N_NODES, D)

    h1, q2, k2, v2 = _tc_res3(feat1, ent_table, Wq2, Wk2, Wv2, bm=1000)

    feat2 = _sc_layer(src, dst, None,
                      _split_heads(q2, NPAD), _split_heads(k2, NPAD),
                      _split_heads(v2, NPAD),
                      None, with_rel=False)
    feat2 = jnp.swapaxes(feat2[:, :N_NODES], 0, 1).reshape(N_NODES, D)

    return _tc_res(feat2, h1, bm=1000)
